# baseline (device time: 96846 ns/iter reference)
import functools

import jax
import jax.numpy as jnp
from jax import lax
from jax.experimental import pallas as pl
from jax.experimental.pallas import tpu as pltpu

N_DEV = 32
E_PER = 4
N_TOK = 2048
D = 512
H = 1024
N_PLANE = 8
N_Z = 4
N_Y = 4
HALF = N_TOK // 2
STRIP = HALF // N_Y
SUB = STRIP // N_Z


def _moe_body(x_ref, ridx_ref, w_ref, out_ref, obuf, wbuf,
              xstage, ystage, zstage,
              x_rs_send, x_rs_recv, y_rs_send, y_rs_recv,
              z_rs_send, z_rs_recv, z_ag_send, z_ag_recv,
              y_ag_send, y_ag_recv, x_ag_send, x_ag_recv):
    my = lax.axis_index("i")
    zz = lax.div(my, N_PLANE)
    q = lax.rem(my, N_PLANE)
    xc = lax.rem(lax.div(q + 1, 2), 2)
    yy = lax.div(q, 2)
    parity = lax.rem(q, 2)
    x_partner = zz * N_PLANE + jnp.bitwise_xor(q, 1)
    y_next = zz * N_PLANE + lax.rem(q + jnp.where(parity == 0, 3, 1), N_PLANE)
    y_prev = zz * N_PLANE + lax.rem(q + jnp.where(parity == 0, 7, 5), N_PLANE)
    zdev = [lax.rem(my + (1 + j) * N_PLANE, N_DEV) for j in range(N_Z - 1)]

    barrier = pltpu.get_barrier_semaphore()
    for nbr in (x_partner, y_next, y_prev, *zdev):
        pl.semaphore_signal(barrier, inc=1, device_id=(nbr,),
                            device_id_type=pl.DeviceIdType.MESH)
    pl.semaphore_wait(barrier, 6)

    for k in range(E_PER):
        wbuf[k, :, :] = w_ref[k].astype(jnp.bfloat16)

    def compute_strip(c):
        sl = pl.ds(c * STRIP, STRIP)
        xrows = x_ref[sl, :].astype(jnp.bfloat16)
        rc = ridx_ref[sl, :]
        acc = jnp.zeros((STRIP, H), jnp.float32)
        for k in range(E_PER):
            mask = (rc == E_PER * my + k).astype(jnp.bfloat16)
            acc = acc + jnp.dot(xrows * mask, wbuf[k],
                                preferred_element_type=jnp.float32)
        obuf[sl, :] = acc.astype(jnp.bfloat16)

    ops = []

    def remote_copy(src_sl, dst_ref, send_sem, recv_sem, target):
        op = pltpu.make_async_remote_copy(
            src_ref=obuf.at[src_sl],
            dst_ref=dst_ref,
            send_sem=send_sem,
            recv_sem=recv_sem,
            device_id=(target,),
            device_id_type=pl.DeviceIdType.MESH,
        )
        op.start()
        ops.append(op)
        return op

    mybase = xc * HALF
    theirbase = (1 - xc) * HALF

    def strip_rows(s):
        return pl.ds(mybase + s * STRIP, STRIP)

    yops = []
    for j in range(N_Y):
        s = lax.rem(yy - j + 2 * N_Y, N_Y)
        compute_strip((1 - xc) * N_Y + s)
        remote_copy(pl.ds(theirbase + s * STRIP, STRIP),
                    xstage.at[pl.ds(s * STRIP, STRIP)],
                    x_rs_send.at[j], x_rs_recv.at[j], x_partner)
        compute_strip(xc * N_Y + s)
        p1 = pltpu.make_async_remote_copy(
            src_ref=xstage.at[pl.ds(0, STRIP)],
            dst_ref=xstage.at[pl.ds(0, STRIP)],
            send_sem=x_rs_send.at[j], recv_sem=x_rs_recv.at[j],
            device_id=(x_partner,), device_id_type=pl.DeviceIdType.MESH,
        )
        p1.wait_recv()
        sl = strip_rows(s)
        obuf[sl, :] = obuf[sl, :] + xstage[pl.ds(s * STRIP, STRIP)]
        if j >= 1:
            yops[j - 1].wait_recv()
            obuf[sl, :] = obuf[sl, :] + ystage[j - 1]
        if j < N_Y - 1:
            yops.append(remote_copy(sl, ystage.at[j],
                                    y_rs_send.at[j], y_rs_recv.at[j],
                                    y_next))
    S = lax.rem(yy + 1, N_Y)

    def sub_rows(s):
        return pl.ds(mybase + S * STRIP + s * SUB, SUB)

    def slot_on(t):
        return jnp.where(zz > t, zz - 1, zz)

    for j in range(N_Z - 1):
        t = lax.rem(zz + 1 + j, N_Z)
        sl_t = slot_on(t)
        remote_copy(sub_rows(t), zstage.at[sl_t],
                    z_rs_send.at[j], z_rs_recv.at[sl_t], zdev[j])
    for i in range(N_Z - 1):
        rcv = pltpu.make_async_remote_copy(
            src_ref=zstage.at[i], dst_ref=zstage.at[i],
            send_sem=z_rs_send.at[i], recv_sem=z_rs_recv.at[i],
            device_id=(zdev[0],), device_id_type=pl.DeviceIdType.MESH,
        )
        rcv.wait_recv()
    sl = sub_rows(zz)
    obuf[sl, :] = obuf[sl, :] + zstage[0] + zstage[1] + zstage[2]

    for j in range(N_Z - 1):
        t = lax.rem(zz + 1 + j, N_Z)
        remote_copy(sub_rows(zz), obuf.at[sub_rows(zz)],
                    z_ag_send.at[j], z_ag_recv.at[slot_on(t)], zdev[j])
    for i in range(N_Z - 1):
        s_i = jnp.where(i >= zz, i + 1, i)
        rcv = pltpu.make_async_remote_copy(
            src_ref=obuf.at[sub_rows(s_i)], dst_ref=obuf.at[sub_rows(s_i)],
            send_sem=z_ag_send.at[i], recv_sem=z_ag_recv.at[i],
            device_id=(zdev[0],), device_id_type=pl.DeviceIdType.MESH,
        )
        rcv.wait_recv()

    sl = strip_rows(S)
    remote_copy(sl, obuf.at[sl], x_ag_send.at[0], x_ag_recv.at[0],
                x_partner)
    for h in range(N_Y - 1):
        s = lax.rem(yy + 1 - h + 2 * N_Y, N_Y)
        sl = strip_rows(s)
        op = remote_copy(sl, obuf.at[sl],
                         y_ag_send.at[h], y_ag_recv.at[h], y_next)
        op.wait_recv()
        rl = strip_rows(lax.rem(yy - h + 2 * N_Y, N_Y))
        remote_copy(rl, obuf.at[rl],
                    x_ag_send.at[h + 1], x_ag_recv.at[h + 1], x_partner)
    sl = pl.ds(mybase, HALF)
    out_ref[sl, :] = obuf[sl, :].astype(jnp.float32)
    for i in range(N_Y):
        rcv = pltpu.make_async_remote_copy(
            src_ref=xstage.at[pl.ds(0, STRIP)],
            dst_ref=xstage.at[pl.ds(0, STRIP)],
            send_sem=x_ag_send.at[i], recv_sem=x_ag_recv.at[i],
            device_id=(x_partner,), device_id_type=pl.DeviceIdType.MESH,
        )
        rcv.wait_recv()
        s_i = lax.rem(yy + 1 - i + 2 * N_Y, N_Y)
        rl = pl.ds(theirbase + s_i * STRIP, STRIP)
        out_ref[rl, :] = obuf[rl, :].astype(jnp.float32)

    for op in ops:
        op.wait_send()

    @functools.partial(pl.run_scoped, sem=pltpu.SemaphoreType.REGULAR)
    def _(sem):
        for nbr in (x_partner, y_next, y_prev, *zdev):
            pl.semaphore_signal(sem, inc=1, device_id=(nbr,),
                                device_id_type=pl.DeviceIdType.MESH)
        pl.semaphore_wait(sem, 6)


def kernel(x, router_W, route_idx, expert_W):
    del router_W
    ny1 = N_Y - 1
    nz1 = N_Z - 1
    dma = pltpu.SemaphoreType.DMA
    return pl.pallas_call(
        _moe_body,
        out_shape=jax.ShapeDtypeStruct((N_TOK, H), jnp.float32),
        in_specs=[
            pl.BlockSpec(memory_space=pltpu.VMEM),
            pl.BlockSpec(memory_space=pltpu.VMEM),
            pl.BlockSpec(memory_space=pltpu.VMEM),
        ],
        out_specs=pl.BlockSpec(memory_space=pltpu.VMEM),
        scratch_shapes=[
            pltpu.VMEM((N_TOK, H), jnp.bfloat16),
            pltpu.VMEM((E_PER, D, H), jnp.bfloat16),
            pltpu.VMEM((HALF, H), jnp.bfloat16),
            pltpu.VMEM((ny1, STRIP, H), jnp.bfloat16),
            pltpu.VMEM((nz1, SUB, H), jnp.bfloat16),
            dma((N_Y,)), dma((N_Y,)),
            dma((ny1,)), dma((ny1,)),
            dma((nz1,)), dma((nz1,)),
            dma((nz1,)), dma((nz1,)),
            dma((ny1,)), dma((ny1,)),
            dma((N_Y,)), dma((N_Y,)),
        ],
        compiler_params=pltpu.CompilerParams(collective_id=0),
    )(x, route_idx.astype(jnp.int32), expert_W)
